# baseline (speedup 1.0000x reference)
import functools

import jax
import jax.numpy as jnp
from jax import lax
from jax.experimental import pallas as pl
from jax.experimental.pallas import tpu as pltpu
from jax.experimental.pallas import tpu_sc as plsc

F32 = jnp.float32
I32 = jnp.int32


def _enc_dist_body(nk, kb, patches_ref, We_ref, be_ref, Wqh_ref, bqh_ref,
                   Wlh_ref, blh_ref, cb_ref, logit_ref, code_ref, zl_ref,
                   zq_s, bv_s, bi_s):
    k = pl.program_id(1)
    tn = patches_ref.shape[0]

    @pl.when(k == 0)
    def _encoder():
        z = jnp.dot(patches_ref[...], We_ref[...],
                    preferred_element_type=F32) + be_ref[...]
        z = jnp.maximum(z, 0.0)
        zq_s[...] = jnp.dot(z, Wqh_ref[...], preferred_element_type=F32) + bqh_ref[...]
        zl_ref[...] = jnp.dot(z, Wlh_ref[...], preferred_element_type=F32) + blh_ref[...]
        bv_s[...] = jnp.full((tn, 1), -jnp.inf, F32)

    zq = zq_s[...]
    cb = cb_ref[...]
    rn = jnp.sum(zq * zq, axis=1, keepdims=True)
    cn = jnp.sum(cb * cb, axis=1)
    prod = lax.dot_general(zq, cb, (((1,), (1,)), ((), ())),
                           preferred_element_type=F32)
    logit = 2.0 * prod - rn - cn[None, :]
    logit_ref[...] = logit

    bmax = jnp.max(logit, axis=1, keepdims=True)
    barg = jnp.argmax(logit, axis=1).astype(I32)[:, None] + k * kb
    better = bmax > bv_s[...]
    bv_s[...] = jnp.where(better, bmax, bv_s[...])
    bi_s[...] = jnp.where(better, barg, bi_s[...])

    @pl.when(k == nk - 1)
    def _emit_code():
        code_ref[...] = bi_s[...]


def _enc_dist(patches, We2, be2, Wqh2, bqh2, Wlh2, blh2, codebook,
              tn=448, kb=2048, interpret=False):
    n, pdim = patches.shape
    K, Dc = codebook.shape
    Cz = We2.shape[1]
    nt, nk = n // tn, K // kb
    grid = (nt, nk)
    out_shapes = (
        jax.ShapeDtypeStruct((n, K), F32),
        jax.ShapeDtypeStruct((n, 1), I32),
        jax.ShapeDtypeStruct((n, Cz), F32),
    )
    return pl.pallas_call(
        functools.partial(_enc_dist_body, nk, kb),
        grid=grid,
        in_specs=[
            pl.BlockSpec((tn, pdim), lambda t, k: (t, 0)),
            pl.BlockSpec((pdim, Cz), lambda t, k: (0, 0)),
            pl.BlockSpec((1, Cz), lambda t, k: (0, 0)),
            pl.BlockSpec((Cz, Dc), lambda t, k: (0, 0)),
            pl.BlockSpec((1, Dc), lambda t, k: (0, 0)),
            pl.BlockSpec((Cz, Cz), lambda t, k: (0, 0)),
            pl.BlockSpec((1, Cz), lambda t, k: (0, 0)),
            pl.BlockSpec((kb, Dc), lambda t, k: (k, 0)),
        ],
        out_specs=(
            pl.BlockSpec((tn, kb), lambda t, k: (t, k)),
            pl.BlockSpec((tn, 1), lambda t, k: (t, 0)),
            pl.BlockSpec((tn, Cz), lambda t, k: (t, 0)),
        ),
        out_shape=out_shapes,
        scratch_shapes=[
            pltpu.VMEM((tn, Dc), F32),
            pltpu.VMEM((tn, 1), F32),
            pltpu.VMEM((tn, 1), I32),
        ],
        interpret=interpret,
    )(patches, We2, be2, Wqh2, bqh2, Wlh2, blh2, codebook)


def _sc_gather(codebook, code_padded):
    K, Dc = codebook.shape
    npad = code_padded.shape[0]
    info = plsc.get_sparse_core_info()
    nw = info.num_cores * info.num_subcores
    b_per_w = npad // nw
    mesh = plsc.VectorSubcoreMesh(core_axis_name="c", subcore_axis_name="s")

    @functools.partial(
        pl.kernel, mesh=mesh,
        out_type=jax.ShapeDtypeStruct((npad, Dc), F32),
        scratch_types=[
            pltpu.VMEM((b_per_w,), I32),
            pltpu.VMEM((b_per_w, Dc), F32),
            pltpu.SemaphoreType.DMA,
        ],
    )
    def gather_k(table_hbm, idx_hbm, out_hbm, idx_v, rows_v, sem):
        wid = lax.axis_index("s") * info.num_cores + lax.axis_index("c")
        base = wid * b_per_w
        pltpu.sync_copy(idx_hbm.at[pl.ds(base, b_per_w)], idx_v)
        pltpu.async_copy(table_hbm.at[idx_v], rows_v, sem).wait()
        pltpu.sync_copy(rows_v, out_hbm.at[pl.ds(base, b_per_w)])

    return gather_k(codebook, code_padded)


def _residual_body(q_ref, zl_ref, Wd_ref, bd_ref, out_ref):
    deq = jnp.dot(q_ref[...], Wd_ref[...], preferred_element_type=F32) + bd_ref[...]
    out_ref[...] = zl_ref[...] - deq


def _residual(q_flat, zl, Wdeq2, bdeq2, tn=448, interpret=False):
    n, Dc = q_flat.shape
    Cz = zl.shape[1]
    return pl.pallas_call(
        _residual_body,
        grid=(n // tn,),
        in_specs=[
            pl.BlockSpec((tn, Dc), lambda t: (t, 0)),
            pl.BlockSpec((tn, Cz), lambda t: (t, 0)),
            pl.BlockSpec((Dc, Cz), lambda t: (0, 0)),
            pl.BlockSpec((1, Cz), lambda t: (0, 0)),
        ],
        out_specs=pl.BlockSpec((tn, Cz), lambda t: (t, 0)),
        out_shape=jax.ShapeDtypeStruct((n, Cz), F32),
        interpret=interpret,
    )(q_flat, zl, Wdeq2, bdeq2)


def kernel(x, W_enc, b_enc, W_qh, b_qh, codebook, W_deq, b_deq, W_lh, b_lh):
    B, Cin, H, W = x.shape
    Cz = W_enc.shape[0]
    K, Dc = codebook.shape
    Ho, Wo = H // 2, W // 2
    N = B * Ho * Wo

    xp = jnp.pad(x, ((0, 0), (0, 0), (1, 1), (1, 1)))
    taps = [xp[:, :, di:di + 2 * Ho - 1:2, dj:dj + 2 * Wo - 1:2]
            for di in range(3) for dj in range(3)]
    pt = jnp.stack(taps, axis=2)
    patches = pt.transpose(0, 3, 4, 1, 2).reshape(N, Cin * 9)

    We2 = W_enc.reshape(Cz, Cin * 9).T
    be2 = b_enc[None, :]
    Wqh2 = W_qh.reshape(Dc, Cz).T
    bqh2 = b_qh[None, :]
    Wlh2 = W_lh.reshape(Cz, Cz).T
    blh2 = b_lh[None, :]
    Wdeq2 = W_deq.reshape(Cz, Dc).T
    bdeq2 = b_deq[None, :]

    logit_flat, code2, zl = _enc_dist(patches, We2, be2, Wqh2, bqh2,
                                      Wlh2, blh2, codebook)
    code_flat = code2[:, 0]

    npad = ((N + 255) // 256) * 256
    code_padded = jnp.concatenate(
        [code_flat, jnp.zeros((npad - N,), I32)]) if npad != N else code_flat
    q_pad = _sc_gather(codebook, code_padded)
    q_flat = q_pad[:N]

    residual_flat = _residual(q_flat, zl, Wdeq2, bdeq2)

    q = q_flat.reshape(B, Ho, Wo, Dc).transpose(0, 3, 1, 2)
    residual = residual_flat.reshape(B, Ho, Wo, Cz).transpose(0, 3, 1, 2)
    code = code_flat.reshape(B, Ho, Wo)
    logit = logit_flat.reshape(B, Ho, Wo, K)
    return (q, residual, code, logit)


# speedup vs baseline: 1.0147x; 1.0147x over previous
import functools

import jax
import jax.numpy as jnp
from jax import lax
from jax.experimental import pallas as pl
from jax.experimental.pallas import tpu as pltpu
from jax.experimental.pallas import tpu_sc as plsc

F32 = jnp.float32
I32 = jnp.int32


def _enc_dist_body(nk, kb, pt_ref, We_ref, be_ref, Wqh_ref, bqh_ref,
                   Wlh_ref, blh_ref, cb_ref, logit_ref, code_ref, zlT_ref,
                   zq_s, bv_s, bi_s):
    k = pl.program_id(1)
    tn = zq_s.shape[0]

    @pl.when(k == 0)
    def _encoder():
        zT = jnp.dot(We_ref[...], pt_ref[0],
                     preferred_element_type=F32) + be_ref[...]
        zT = jnp.maximum(zT, 0.0)
        zqT = jnp.dot(Wqh_ref[...], zT, preferred_element_type=F32) + bqh_ref[...]
        zlT_ref[0] = jnp.dot(Wlh_ref[...], zT, preferred_element_type=F32) + blh_ref[...]
        zq_s[...] = jnp.swapaxes(zqT, 0, 1)
        bv_s[...] = jnp.full((tn, 1), -jnp.inf, F32)

    zq = zq_s[...]
    cb = cb_ref[...]
    rn = jnp.sum(zq * zq, axis=1, keepdims=True)
    cn = jnp.sum(cb * cb, axis=1)
    prod = lax.dot_general(zq, cb, (((1,), (1,)), ((), ())),
                           preferred_element_type=F32)
    logit = 2.0 * prod - rn - cn[None, :]
    logit_ref[...] = logit

    bmax = jnp.max(logit, axis=1, keepdims=True)
    barg = jnp.argmax(logit, axis=1).astype(I32)[:, None] + k * kb
    better = bmax > bv_s[...]
    bv_s[...] = jnp.where(better, bmax, bv_s[...])
    bi_s[...] = jnp.where(better, barg, bi_s[...])

    @pl.when(k == nk - 1)
    def _emit_code():
        code_ref[...] = bi_s[...]


def _enc_dist(patchesT, We, be, Wqh, bqh, Wlh, blh, codebook,
              kb=2048, interpret=False):
    B, pdim, tn = patchesT.shape
    K, Dc = codebook.shape
    Cz = We.shape[0]
    n = B * tn
    nk = K // kb
    out_shapes = (
        jax.ShapeDtypeStruct((n, K), F32),
        jax.ShapeDtypeStruct((n, 1), I32),
        jax.ShapeDtypeStruct((B, Cz, tn), F32),
    )
    return pl.pallas_call(
        functools.partial(_enc_dist_body, nk, kb),
        grid=(B, nk),
        in_specs=[
            pl.BlockSpec((1, pdim, tn), lambda b, k: (b, 0, 0)),
            pl.BlockSpec((Cz, pdim), lambda b, k: (0, 0)),
            pl.BlockSpec((Cz, 1), lambda b, k: (0, 0)),
            pl.BlockSpec((Dc, Cz), lambda b, k: (0, 0)),
            pl.BlockSpec((Dc, 1), lambda b, k: (0, 0)),
            pl.BlockSpec((Cz, Cz), lambda b, k: (0, 0)),
            pl.BlockSpec((Cz, 1), lambda b, k: (0, 0)),
            pl.BlockSpec((kb, Dc), lambda b, k: (k, 0)),
        ],
        out_specs=(
            pl.BlockSpec((tn, kb), lambda b, k: (b, k)),
            pl.BlockSpec((tn, 1), lambda b, k: (b, 0)),
            pl.BlockSpec((1, Cz, tn), lambda b, k: (b, 0, 0)),
        ),
        out_shape=out_shapes,
        scratch_shapes=[
            pltpu.VMEM((tn, Dc), F32),
            pltpu.VMEM((tn, 1), F32),
            pltpu.VMEM((tn, 1), I32),
        ],
        interpret=interpret,
    )(patchesT, We, be, Wqh, bqh, Wlh, blh, codebook)


def _sc_gather(codebook, code_padded):
    K, Dc = codebook.shape
    npad = code_padded.shape[0]
    info = plsc.get_sparse_core_info()
    nw = info.num_cores * info.num_subcores
    b_per_w = npad // nw
    mesh = plsc.VectorSubcoreMesh(core_axis_name="c", subcore_axis_name="s")

    @functools.partial(
        pl.kernel, mesh=mesh,
        out_type=jax.ShapeDtypeStruct((npad, Dc), F32),
        scratch_types=[
            pltpu.VMEM((b_per_w,), I32),
            pltpu.VMEM((b_per_w, Dc), F32),
            pltpu.SemaphoreType.DMA,
        ],
    )
    def gather_k(table_hbm, idx_hbm, out_hbm, idx_v, rows_v, sem):
        wid = lax.axis_index("s") * info.num_cores + lax.axis_index("c")
        base = wid * b_per_w
        pltpu.sync_copy(idx_hbm.at[pl.ds(base, b_per_w)], idx_v)
        pltpu.async_copy(table_hbm.at[idx_v], rows_v, sem).wait()
        pltpu.sync_copy(rows_v, out_hbm.at[pl.ds(base, b_per_w)])

    return gather_k(codebook, code_padded)


def _residual_body(q_ref, zlT_ref, Wd_ref, bd_ref, out_ref, qT_ref):
    qT = jnp.swapaxes(q_ref[...], 0, 1)
    qT_ref[0] = qT
    deqT = jnp.dot(Wd_ref[...], qT, preferred_element_type=F32) + bd_ref[...]
    out_ref[0] = zlT_ref[0] - deqT


def _residual(q_flat, zlT, Wd, bd, interpret=False):
    n, Dc = q_flat.shape
    B, Cz, tn = zlT.shape
    return pl.pallas_call(
        _residual_body,
        grid=(B,),
        in_specs=[
            pl.BlockSpec((tn, Dc), lambda b: (b, 0)),
            pl.BlockSpec((1, Cz, tn), lambda b: (b, 0, 0)),
            pl.BlockSpec((Cz, Dc), lambda b: (0, 0)),
            pl.BlockSpec((Cz, 1), lambda b: (0, 0)),
        ],
        out_specs=(
            pl.BlockSpec((1, Cz, tn), lambda b: (b, 0, 0)),
            pl.BlockSpec((1, Dc, tn), lambda b: (b, 0, 0)),
        ),
        out_shape=(
            jax.ShapeDtypeStruct((B, Cz, tn), F32),
            jax.ShapeDtypeStruct((B, Dc, tn), F32),
        ),
        interpret=interpret,
    )(q_flat, zlT, Wd, bd)


def kernel(x, W_enc, b_enc, W_qh, b_qh, codebook, W_deq, b_deq, W_lh, b_lh):
    B, Cin, H, W = x.shape
    Cz = W_enc.shape[0]
    K, Dc = codebook.shape
    Ho, Wo = H // 2, W // 2
    N = B * Ho * Wo
    tn = Ho * Wo

    xp = jnp.pad(x, ((0, 0), (0, 0), (1, 1), (1, 1)))
    taps = [xp[:, :, di:di + 2 * Ho - 1:2, dj:dj + 2 * Wo - 1:2]
            for di in range(3) for dj in range(3)]
    patchesT = jnp.stack(taps, axis=2).reshape(B, Cin * 9, tn)

    We = W_enc.reshape(Cz, Cin * 9)
    Wqh = W_qh.reshape(Dc, Cz)
    Wlh = W_lh.reshape(Cz, Cz)
    Wd = W_deq.reshape(Cz, Dc)
    be, bqh, blh, bd = (b_enc[:, None], b_qh[:, None],
                        b_lh[:, None], b_deq[:, None])

    logit_flat, code2, zlT = _enc_dist(patchesT, We, be, Wqh, bqh,
                                       Wlh, blh, codebook)
    code_flat = code2[:, 0]

    npad = ((N + 255) // 256) * 256
    code_padded = jnp.concatenate(
        [code_flat, jnp.zeros((npad - N,), I32)]) if npad != N else code_flat
    q_pad = _sc_gather(codebook, code_padded)
    q_flat = q_pad[:N]

    residualT, qT = _residual(q_flat, zlT, Wd, bd)

    q = qT.reshape(B, Dc, Ho, Wo)
    residual = residualT.reshape(B, Cz, Ho, Wo)
    code = code_flat.reshape(B, Ho, Wo)
    logit = logit_flat.reshape(B, Ho, Wo, K)
    return (q, residual, code, logit)


# speedup vs baseline: 2.1165x; 2.0859x over previous
import functools

import jax
import jax.numpy as jnp
import numpy as np
from jax import lax
from jax.experimental import pallas as pl
from jax.experimental.pallas import tpu as pltpu
from jax.experimental.pallas import tpu_sc as plsc

F32 = jnp.float32
I32 = jnp.int32

_GROUPS = (
    ((0, 0), ((0, 0), (0, 1), (1, 0), (1, 1))),
    ((0, 1), ((0, 2), (1, 2))),
    ((1, 0), ((2, 0), (2, 1))),
    ((1, 1), ((2, 2),)),
)


def _enc_dist_body(nk, kb, p00, p01, p10, p11, wr, be, wqh, bqh, wlh, blh,
                   s00, s01, s10, s11, cb_ref, logit_ref, code_ref, zlT_ref,
                   zq_s, bv_s, bi_s):
    k = pl.program_id(1)
    tn = zq_s.shape[0]
    Cz = wlh.shape[0]

    @pl.when(k == 0)
    def _encoder():
        P = {(0, 0): p00, (0, 1): p01, (1, 0): p10, (1, 1): p11}
        S = {(0, 0): s00, (0, 1): s01, (1, 0): s10, (1, 1): s11}
        zpre = be[...] + jnp.zeros((Cz, tn), F32)
        for (r0, c0), taps in _GROUPS:
            U = None
            for (di, dj) in taps:
                t = di * 3 + dj
                Wt = wr[pl.ds(t * Cz, Cz), :]
                term = jnp.dot(Wt, P[(di % 2, dj % 2)][0],
                               preferred_element_type=F32)
                U = term if U is None else U + term
            zpre = zpre + jnp.dot(U, S[(r0, c0)][...],
                                  preferred_element_type=F32,
                                  precision=lax.Precision.HIGHEST)
        zT = jnp.maximum(zpre, 0.0)
        zqT = jnp.dot(wqh[...], zT, preferred_element_type=F32) + bqh[...]
        zlT_ref[0] = jnp.dot(wlh[...], zT, preferred_element_type=F32) + blh[...]
        zq_s[...] = jnp.swapaxes(zqT, 0, 1)
        bv_s[...] = jnp.full((tn, 1), -jnp.inf, F32)

    zq = zq_s[...]
    cb = cb_ref[...]
    rn = jnp.sum(zq * zq, axis=1, keepdims=True)
    cn = jnp.sum(cb * cb, axis=1)
    prod = lax.dot_general(zq, cb, (((1,), (1,)), ((), ())),
                           preferred_element_type=F32)
    logit = 2.0 * prod - rn - cn[None, :]
    logit_ref[...] = logit

    bmax = jnp.max(logit, axis=1, keepdims=True)
    barg = jnp.argmax(logit, axis=1).astype(I32)[:, None] + k * kb
    better = bmax > bv_s[...]
    bv_s[...] = jnp.where(better, bmax, bv_s[...])
    bi_s[...] = jnp.where(better, barg, bi_s[...])

    @pl.when(k == nk - 1)
    def _emit_code():
        code_ref[...] = bi_s[...]


def _enc_dist(planes, wr, be, wqh, bqh, wlh, blh, sels, codebook,
              tn, kb=2048, interpret=False):
    B, Cin, lp = planes[0].shape
    K, Dc = codebook.shape
    Cz = wlh.shape[0]
    n = B * tn
    nk = K // kb
    plane_spec = pl.BlockSpec((1, Cin, lp), lambda b, k: (b, 0, 0))
    const2 = lambda shape: pl.BlockSpec(shape, lambda b, k: (0, 0))
    return pl.pallas_call(
        functools.partial(_enc_dist_body, nk, kb),
        grid=(B, nk),
        in_specs=[plane_spec] * 4 + [
            const2((9 * Cz, Cin)),
            const2((Cz, 1)),
            const2((Dc, Cz)),
            const2((Dc, 1)),
            const2((Cz, Cz)),
            const2((Cz, 1)),
            const2((lp, tn)),
            const2((lp, tn)),
            const2((lp, tn)),
            const2((lp, tn)),
            pl.BlockSpec((kb, Dc), lambda b, k: (k, 0)),
        ],
        out_specs=(
            pl.BlockSpec((tn, kb), lambda b, k: (b, k)),
            pl.BlockSpec((tn, 1), lambda b, k: (b, 0)),
            pl.BlockSpec((1, Cz, tn), lambda b, k: (b, 0, 0)),
        ),
        out_shape=(
            jax.ShapeDtypeStruct((n, K), F32),
            jax.ShapeDtypeStruct((n, 1), I32),
            jax.ShapeDtypeStruct((B, Cz, tn), F32),
        ),
        scratch_shapes=[
            pltpu.VMEM((tn, Dc), F32),
            pltpu.VMEM((tn, 1), F32),
            pltpu.VMEM((tn, 1), I32),
        ],
        interpret=interpret,
    )(*planes, wr, be, wqh, bqh, wlh, blh, *sels, codebook)


def _sc_gather(codebook, code_padded):
    K, Dc = codebook.shape
    npad = code_padded.shape[0]
    info = plsc.get_sparse_core_info()
    nw = info.num_cores * info.num_subcores
    b_per_w = npad // nw
    mesh = plsc.VectorSubcoreMesh(core_axis_name="c", subcore_axis_name="s")

    @functools.partial(
        pl.kernel, mesh=mesh,
        out_type=jax.ShapeDtypeStruct((npad, Dc), F32),
        scratch_types=[
            pltpu.VMEM((b_per_w,), I32),
            pltpu.VMEM((b_per_w, Dc), F32),
            pltpu.SemaphoreType.DMA,
        ],
    )
    def gather_k(table_hbm, idx_hbm, out_hbm, idx_v, rows_v, sem):
        wid = lax.axis_index("s") * info.num_cores + lax.axis_index("c")
        base = wid * b_per_w
        pltpu.sync_copy(idx_hbm.at[pl.ds(base, b_per_w)], idx_v)
        pltpu.async_copy(table_hbm.at[idx_v], rows_v, sem).wait()
        pltpu.sync_copy(rows_v, out_hbm.at[pl.ds(base, b_per_w)])

    return gather_k(codebook, code_padded)


def _residual_body(q_ref, zlT_ref, Wd_ref, bd_ref, out_ref, qT_ref):
    qT = jnp.swapaxes(q_ref[...], 0, 1)
    qT_ref[0] = qT
    deqT = jnp.dot(Wd_ref[...], qT, preferred_element_type=F32) + bd_ref[...]
    out_ref[0] = zlT_ref[0] - deqT


def _residual(q_flat, zlT, Wd, bd, interpret=False):
    n, Dc = q_flat.shape
    B, Cz, tn = zlT.shape
    return pl.pallas_call(
        _residual_body,
        grid=(B,),
        in_specs=[
            pl.BlockSpec((tn, Dc), lambda b: (b, 0)),
            pl.BlockSpec((1, Cz, tn), lambda b: (b, 0, 0)),
            pl.BlockSpec((Cz, Dc), lambda b: (0, 0)),
            pl.BlockSpec((Cz, 1), lambda b: (0, 0)),
        ],
        out_specs=(
            pl.BlockSpec((1, Cz, tn), lambda b: (b, 0, 0)),
            pl.BlockSpec((1, Dc, tn), lambda b: (b, 0, 0)),
        ),
        out_shape=(
            jax.ShapeDtypeStruct((B, Cz, tn), F32),
            jax.ShapeDtypeStruct((B, Dc, tn), F32),
        ),
        interpret=interpret,
    )(q_flat, zlT, Wd, bd)


def _parity_planes(x):
    B, Cin, H, W = x.shape
    Hh, Wh = H // 2 + 1, W // 2 + 1
    xp = jnp.pad(x, ((0, 0), (0, 0), (1, 1), (1, 1)))
    xr = xp.reshape(B, Cin, H + 2, Wh, 2)
    planes = []
    for pw in range(2):
        a = xr[..., pw].reshape(B, Cin, Hh, 2, Wh)
        for ph in range(2):
            planes.append(a[:, :, :, ph, :].reshape(B, Cin, Hh * Wh))
    p00, p10, p01, p11 = planes
    return p00, p01, p10, p11


def _selection_mats(Ho, Wo):
    Hh, Wh = Ho + 1, Wo + 1
    sels = []
    for r0 in range(2):
        for c0 in range(2):
            s = np.zeros((Hh * Wh, Ho * Wo), np.float32)
            for i in range(Ho):
                s[(np.arange(Wo) + (i + r0) * Wh + c0),
                  np.arange(Wo) + i * Wo] = 1.0
            sels.append(jnp.asarray(s))
    return sels


def kernel(x, W_enc, b_enc, W_qh, b_qh, codebook, W_deq, b_deq, W_lh, b_lh):
    B, Cin, H, W = x.shape
    Cz = W_enc.shape[0]
    K, Dc = codebook.shape
    Ho, Wo = H // 2, W // 2
    N = B * Ho * Wo
    tn = Ho * Wo

    planes = _parity_planes(x)
    sels = _selection_mats(Ho, Wo)

    wr = W_enc.transpose(2, 3, 0, 1).reshape(9 * Cz, Cin)
    wqh = W_qh.reshape(Dc, Cz)
    wlh = W_lh.reshape(Cz, Cz)
    wd = W_deq.reshape(Cz, Dc)
    be, bqh, blh, bd = (b_enc[:, None], b_qh[:, None],
                        b_lh[:, None], b_deq[:, None])

    logit_flat, code2, zlT = _enc_dist(planes, wr, be, wqh, bqh, wlh, blh,
                                       sels, codebook, tn)
    code_flat = code2[:, 0]

    npad = ((N + 255) // 256) * 256
    code_padded = jnp.concatenate(
        [code_flat, jnp.zeros((npad - N,), I32)]) if npad != N else code_flat
    q_pad = _sc_gather(codebook, code_padded)
    q_flat = q_pad[:N]

    residualT, qT = _residual(q_flat, zlT, wd, bd)

    q = qT.reshape(B, Dc, Ho, Wo)
    residual = residualT.reshape(B, Cz, Ho, Wo)
    code = code_flat.reshape(B, Ho, Wo)
    logit = logit_flat.reshape(B, Ho, Wo, K)
    return (q, residual, code, logit)


# speedup vs baseline: 2.4787x; 1.1711x over previous
import functools

import jax
import jax.numpy as jnp
import numpy as np
from jax import lax
from jax.experimental import pallas as pl
from jax.experimental.pallas import tpu as pltpu
from jax.experimental.pallas import tpu_sc as plsc

F32 = jnp.float32
I32 = jnp.int32

_GROUPS = (
    ((0, 0), ((0, 0), (0, 1), (1, 0), (1, 1))),
    ((0, 1), ((0, 2), (1, 2))),
    ((1, 0), ((2, 0), (2, 1))),
    ((1, 1), ((2, 2),)),
)


def _enc_dist_body(nk, kb, p00, p01, p10, p11, wr, be, wqh, bqh, wlh, blh,
                   s00, s01, s10, s11, cb_ref, logit_ref, code_ref, zlT_ref,
                   zq_s, bv_s, bi_s):
    k = pl.program_id(1)
    tn = zq_s.shape[0]
    Cz = wlh.shape[0]

    @pl.when(k == 0)
    def _encoder():
        S = {(0, 0): s00, (0, 1): s01, (1, 0): s10, (1, 1): s11}
        PS = {}
        for key, p in (((0, 0), p00), ((0, 1), p01),
                       ((1, 0), p10), ((1, 1), p11)):
            v = p[0]
            hi = v.astype(jnp.bfloat16)
            lo = (v - hi.astype(F32)).astype(jnp.bfloat16)
            PS[key] = (hi, lo)
        zpre = be[...] + jnp.zeros((Cz, tn), F32)
        for (r0, c0), taps in _GROUPS:
            sel = S[(r0, c0)][...]
            for (di, dj) in taps:
                t = di * 3 + dj
                Wt = wr[pl.ds(t * Cz, Cz), :]
                hi, lo = PS[(di % 2, dj % 2)]
                psel = (jnp.dot(hi, sel, preferred_element_type=F32)
                        + jnp.dot(lo, sel, preferred_element_type=F32))
                zpre = zpre + jnp.dot(Wt, psel, preferred_element_type=F32)
        zT = jnp.maximum(zpre, 0.0)
        zqT = jnp.dot(wqh[...], zT, preferred_element_type=F32) + bqh[...]
        zlT_ref[0] = jnp.dot(wlh[...], zT, preferred_element_type=F32) + blh[...]
        zq_s[...] = jnp.swapaxes(zqT, 0, 1)
        bv_s[...] = jnp.full((tn, 1), -jnp.inf, F32)

    zq = zq_s[...]
    cb = cb_ref[...]
    rn = jnp.sum(zq * zq, axis=1, keepdims=True)
    cn = jnp.sum(cb * cb, axis=1)
    prod = lax.dot_general(zq, cb, (((1,), (1,)), ((), ())),
                           preferred_element_type=F32)
    logit = 2.0 * prod - rn - cn[None, :]
    logit_ref[...] = logit

    bmax = jnp.max(logit, axis=1, keepdims=True)
    barg = jnp.argmax(logit, axis=1).astype(I32)[:, None] + k * kb
    better = bmax > bv_s[...]
    bv_s[...] = jnp.where(better, bmax, bv_s[...])
    bi_s[...] = jnp.where(better, barg, bi_s[...])

    @pl.when(k == nk - 1)
    def _emit_code():
        code_ref[...] = bi_s[...]


def _enc_dist(planes, wr, be, wqh, bqh, wlh, blh, sels, codebook,
              tn, kb=2048, interpret=False):
    B, Cin, lp = planes[0].shape
    K, Dc = codebook.shape
    Cz = wlh.shape[0]
    n = B * tn
    nk = K // kb
    plane_spec = pl.BlockSpec((1, Cin, lp), lambda b, k: (b, 0, 0))
    const2 = lambda shape: pl.BlockSpec(shape, lambda b, k: (0, 0))
    return pl.pallas_call(
        functools.partial(_enc_dist_body, nk, kb),
        grid=(B, nk),
        in_specs=[plane_spec] * 4 + [
            const2((9 * Cz, Cin)),
            const2((Cz, 1)),
            const2((Dc, Cz)),
            const2((Dc, 1)),
            const2((Cz, Cz)),
            const2((Cz, 1)),
            const2((lp, tn)),
            const2((lp, tn)),
            const2((lp, tn)),
            const2((lp, tn)),
            pl.BlockSpec((kb, Dc), lambda b, k: (k, 0)),
        ],
        out_specs=(
            pl.BlockSpec((tn, kb), lambda b, k: (b, k)),
            pl.BlockSpec((tn, 1), lambda b, k: (b, 0)),
            pl.BlockSpec((1, Cz, tn), lambda b, k: (b, 0, 0)),
        ),
        out_shape=(
            jax.ShapeDtypeStruct((n, K), F32),
            jax.ShapeDtypeStruct((n, 1), I32),
            jax.ShapeDtypeStruct((B, Cz, tn), F32),
        ),
        scratch_shapes=[
            pltpu.VMEM((tn, Dc), F32),
            pltpu.VMEM((tn, 1), F32),
            pltpu.VMEM((tn, 1), I32),
        ],
        interpret=interpret,
    )(*planes, wr, be, wqh, bqh, wlh, blh, *sels, codebook)


def _sc_gather(codebook, code_padded):
    K, Dc = codebook.shape
    npad = code_padded.shape[0]
    info = plsc.get_sparse_core_info()
    nw = info.num_cores * info.num_subcores
    b_per_w = npad // nw
    mesh = plsc.VectorSubcoreMesh(core_axis_name="c", subcore_axis_name="s")

    @functools.partial(
        pl.kernel, mesh=mesh,
        out_type=jax.ShapeDtypeStruct((npad, Dc), F32),
        scratch_types=[
            pltpu.VMEM((b_per_w,), I32),
            pltpu.VMEM((b_per_w, Dc), F32),
            pltpu.SemaphoreType.DMA,
        ],
    )
    def gather_k(table_hbm, idx_hbm, out_hbm, idx_v, rows_v, sem):
        wid = lax.axis_index("s") * info.num_cores + lax.axis_index("c")
        base = wid * b_per_w
        pltpu.sync_copy(idx_hbm.at[pl.ds(base, b_per_w)], idx_v)
        pltpu.async_copy(table_hbm.at[idx_v], rows_v, sem).wait()
        pltpu.sync_copy(rows_v, out_hbm.at[pl.ds(base, b_per_w)])

    return gather_k(codebook, code_padded)


def _residual_body(q_ref, zlT_ref, Wd_ref, bd_ref, out_ref, qT_ref):
    qT = jnp.swapaxes(q_ref[...], 0, 1)
    qT_ref[0] = qT
    deqT = jnp.dot(Wd_ref[...], qT, preferred_element_type=F32) + bd_ref[...]
    out_ref[0] = zlT_ref[0] - deqT


def _residual(q_pad, zlT, Wd, bd, interpret=False):
    npad, Dc = q_pad.shape
    B, Cz, tn = zlT.shape
    return pl.pallas_call(
        _residual_body,
        grid=(B,),
        in_specs=[
            pl.BlockSpec((tn, Dc), lambda b: (b, 0)),
            pl.BlockSpec((1, Cz, tn), lambda b: (b, 0, 0)),
            pl.BlockSpec((Cz, Dc), lambda b: (0, 0)),
            pl.BlockSpec((Cz, 1), lambda b: (0, 0)),
        ],
        out_specs=(
            pl.BlockSpec((1, Cz, tn), lambda b: (b, 0, 0)),
            pl.BlockSpec((1, Dc, tn), lambda b: (b, 0, 0)),
        ),
        out_shape=(
            jax.ShapeDtypeStruct((B, Cz, tn), F32),
            jax.ShapeDtypeStruct((B, Dc, tn), F32),
        ),
        interpret=interpret,
    )(q_pad, zlT, Wd, bd)


def _parity_planes(x):
    B, Cin, H, W = x.shape
    Hh, Wh = H // 2 + 1, W // 2 + 1
    xp = jnp.pad(x, ((0, 0), (0, 0), (1, 1), (1, 1)))
    xr = xp.reshape(B, Cin, H + 2, Wh, 2)
    planes = []
    for pw in range(2):
        a = xr[..., pw].reshape(B, Cin, Hh, 2, Wh)
        for ph in range(2):
            planes.append(a[:, :, :, ph, :].reshape(B, Cin, Hh * Wh))
    p00, p10, p01, p11 = planes
    return p00, p01, p10, p11


def _selection_mats(Ho, Wo):
    Hh, Wh = Ho + 1, Wo + 1
    sels = []
    for r0 in range(2):
        for c0 in range(2):
            s = np.zeros((Hh * Wh, Ho * Wo), np.float32)
            for i in range(Ho):
                s[(np.arange(Wo) + (i + r0) * Wh + c0),
                  np.arange(Wo) + i * Wo] = 1.0
            sels.append(jnp.asarray(s, dtype=jnp.bfloat16))
    return sels


def kernel(x, W_enc, b_enc, W_qh, b_qh, codebook, W_deq, b_deq, W_lh, b_lh):
    B, Cin, H, W = x.shape
    Cz = W_enc.shape[0]
    K, Dc = codebook.shape
    Ho, Wo = H // 2, W // 2
    N = B * Ho * Wo
    tn = Ho * Wo

    planes = _parity_planes(x)
    sels = _selection_mats(Ho, Wo)

    wr = W_enc.transpose(2, 3, 0, 1).reshape(9 * Cz, Cin)
    wqh = W_qh.reshape(Dc, Cz)
    wlh = W_lh.reshape(Cz, Cz)
    wd = W_deq.reshape(Cz, Dc)
    be, bqh, blh, bd = (b_enc[:, None], b_qh[:, None],
                        b_lh[:, None], b_deq[:, None])

    logit_flat, code2, zlT = _enc_dist(planes, wr, be, wqh, bqh, wlh, blh,
                                       sels, codebook, tn)
    code_flat = code2[:, 0]

    npad = ((N + 255) // 256) * 256
    code_padded = jnp.concatenate(
        [code_flat, jnp.zeros((npad - N,), I32)]) if npad != N else code_flat
    q_pad = _sc_gather(codebook, code_padded)

    residualT, qT = _residual(q_pad, zlT, wd, bd)

    q = qT.reshape(B, Dc, Ho, Wo)
    residual = residualT.reshape(B, Cz, Ho, Wo)
    code = code_flat.reshape(B, Ho, Wo)
    logit = logit_flat.reshape(B, Ho, Wo, K)
    return (q, residual, code, logit)
